# Initial kernel scaffold; baseline (speedup 1.0000x reference)
import jax
import jax.numpy as jnp
from jax.experimental import pallas as pl


def kernel(x, edge_index, edge_weight, W1, bias1, W2, bias2):
    raise NotImplementedError("write your pallas kernel here")


# speedup vs baseline: 1.0469x; 1.0469x over previous
import jax
import jax.numpy as jnp
from jax.experimental import pallas as pl


def _mm1_body(x_ref, w_ref, o_ref):
    o_ref[...] = jnp.dot(x_ref[...], w_ref[...], preferred_element_type=jnp.float32)


def _propagate(h, src, dst, w):
    msgs = jnp.take(h, src, axis=0) * w[:, None]
    return jnp.zeros(h.shape, h.dtype).at[dst].add(msgs)


def kernel(x, edge_index, edge_weight, W1, bias1, W2, bias2):
    N = x.shape[0]
    BN = 2000
    h = pl.pallas_call(
        _mm1_body,
        grid=(N // BN,),
        in_specs=[
            pl.BlockSpec((BN, x.shape[1]), lambda i: (i, 0)),
            pl.BlockSpec((x.shape[1], W1.shape[1]), lambda i: (0, 0)),
        ],
        out_specs=pl.BlockSpec((BN, W1.shape[1]), lambda i: (i, 0)),
        out_shape=jax.ShapeDtypeStruct((N, W1.shape[1]), jnp.float32),
    )(x, W1)
    src, dst = edge_index[0], edge_index[1]
    h = jax.nn.relu(_propagate(h, src, dst, edge_weight) + bias1)
    h = h @ W2
    return _propagate(h, src, dst, edge_weight) + bias2


# speedup vs baseline: 2.5178x; 2.4050x over previous
import dataclasses
import functools

import jax
import jax.numpy as jnp
from jax import lax
from jax.experimental import pallas as pl
from jax.experimental.pallas import tpu as pltpu
from jax.experimental.pallas import tpu_sc as plsc

NC, NS, L = 2, 16, 16
NW = NC * NS


def _make_propagate(n, d, e, chunk):
    ep = e // NW
    rpt = n // NS
    zrows = 128
    assert ep % chunk == 0 and rpt % zrows == 0 and d % L == 0
    mesh = plsc.VectorSubcoreMesh(core_axis_name="c", subcore_axis_name="s")
    cp = pltpu.CompilerParams()
    if "needs_layout_passes" in pltpu.CompilerParams.__dataclass_fields__:
        cp = dataclasses.replace(cp, needs_layout_passes=False)

    @functools.partial(
        pl.kernel,
        compiler_params=cp,
        out_type=jax.ShapeDtypeStruct((NC, n, d), jnp.float32),
        mesh=mesh,
        scratch_types=[
            pltpu.VMEM((chunk,), jnp.int32),
            pltpu.VMEM((chunk,), jnp.int32),
            pltpu.VMEM((chunk,), jnp.float32),
            pltpu.VMEM((chunk, d), jnp.float32),
            pltpu.VMEM((zrows, d), jnp.float32),
            pltpu.VMEM_SHARED((n, d), jnp.float32),
            pltpu.SemaphoreType.DMA,
        ],
    )
    def prop(h_hbm, src_hbm, dst_hbm, w_hbm, out_hbm,
             src_v, dst_v, w_v, rows_v, z_v, acc_sh, sem):
        cid = lax.axis_index("c")
        sid = lax.axis_index("s")
        wid = sid * NC + cid
        row0 = sid * rpt

        @pl.loop(0, zrows)
        def _(i):
            for j in range(d // L):
                z_v[i, pl.ds(j * L, L)] = jnp.zeros((L,), jnp.float32)

        for k in range(rpt // zrows):
            pltpu.sync_copy(z_v, acc_sh.at[pl.ds(row0 + k * zrows, zrows)])
        plsc.subcore_barrier()

        base = wid * ep

        @pl.loop(0, ep, step=chunk)
        def _(e0):
            pltpu.sync_copy(src_hbm.at[pl.ds(base + e0, chunk)], src_v)
            pltpu.sync_copy(w_hbm.at[pl.ds(base + e0, chunk)], w_v)
            pltpu.sync_copy(dst_hbm.at[pl.ds(base + e0, chunk)], dst_v)
            pltpu.async_copy(h_hbm.at[src_v], rows_v, sem).wait()

            @pl.loop(0, chunk)
            def _(i):
                wrow = plsc.load_gather(w_v, [jnp.full((L,), i, jnp.int32)])
                for j in range(d // L):
                    sl = pl.ds(j * L, L)
                    rows_v[i, sl] = rows_v[i, sl] * wrow

            pltpu.sync_copy(rows_v, acc_sh.at[dst_v], add=True)

        plsc.subcore_barrier()
        pltpu.sync_copy(acc_sh.at[pl.ds(row0, rpt)],
                        out_hbm.at[cid, pl.ds(row0, rpt)])

    return prop


def _mm1_body(x_ref, w_ref, o_ref):
    o_ref[...] = jnp.dot(x_ref[...], w_ref[...],
                         preferred_element_type=jnp.float32)


def _mid_body(p_ref, b_ref, w_ref, o_ref):
    h = jax.nn.relu(p_ref[0] + p_ref[1] + b_ref[...])
    o_ref[...] = jnp.dot(h, w_ref[...], preferred_element_type=jnp.float32)


def _fin_body(p_ref, b_ref, o_ref):
    c = o_ref.shape[1]
    o_ref[...] = p_ref[0, :, :c] + p_ref[1, :, :c] + b_ref[...]


def kernel(x, edge_index, edge_weight, W1, bias1, W2, bias2):
    n, f_in = x.shape
    hid = W1.shape[1]
    c = W2.shape[1]
    e = edge_weight.shape[0]
    d2 = 128
    np_ = 10240

    src = edge_index[0]
    dst = edge_index[1]
    W2p = jnp.pad(W2, ((0, 0), (0, d2 - c)))
    xp = jnp.pad(x, ((0, np_ - n), (0, 0)))

    bn = 2048
    h1 = pl.pallas_call(
        _mm1_body,
        grid=(np_ // bn,),
        in_specs=[
            pl.BlockSpec((bn, f_in), lambda i: (i, 0)),
            pl.BlockSpec((f_in, hid), lambda i: (0, 0)),
        ],
        out_specs=pl.BlockSpec((bn, hid), lambda i: (i, 0)),
        out_shape=jax.ShapeDtypeStruct((np_, hid), jnp.float32),
    )(xp, W1)

    p1 = _make_propagate(np_, hid, e, 40)(h1, src, dst, edge_weight)

    h2 = pl.pallas_call(
        _mid_body,
        grid=(np_ // bn,),
        in_specs=[
            pl.BlockSpec((NC, bn, hid), lambda i: (0, i, 0)),
            pl.BlockSpec((hid,), lambda i: (0,)),
            pl.BlockSpec((hid, d2), lambda i: (0, 0)),
        ],
        out_specs=pl.BlockSpec((bn, d2), lambda i: (i, 0)),
        out_shape=jax.ShapeDtypeStruct((np_, d2), jnp.float32),
    )(p1, bias1, W2p)

    p2 = _make_propagate(np_, d2, e, 40)(h2, src, dst, edge_weight)

    out = pl.pallas_call(
        _fin_body,
        grid=(np_ // bn,),
        in_specs=[
            pl.BlockSpec((NC, bn, d2), lambda i: (0, i, 0)),
            pl.BlockSpec((c,), lambda i: (0,)),
        ],
        out_specs=pl.BlockSpec((bn, c), lambda i: (i, 0)),
        out_shape=jax.ShapeDtypeStruct((np_, c), jnp.float32),
    )(p2, bias2)
    return out[:n]


# speedup vs baseline: 10.7954x; 4.2876x over previous
import dataclasses
import functools

import jax
import jax.numpy as jnp
from jax import lax
from jax.experimental import pallas as pl
from jax.experimental.pallas import tpu as pltpu
from jax.experimental.pallas import tpu_sc as plsc

NC, NS, L = 2, 16, 16
NW = NC * NS


def _make_propagate(n, d, e, chunk, nb):
    ep = e // NW
    ncl = ep // chunk
    rpt = n // NS
    assert ep % chunk == 0 and ncl % nb == 0 and rpt % chunk == 0 and d % L == 0
    mesh = plsc.VectorSubcoreMesh(core_axis_name="c", subcore_axis_name="s")
    cp = pltpu.CompilerParams()
    if "needs_layout_passes" in pltpu.CompilerParams.__dataclass_fields__:
        cp = dataclasses.replace(cp, needs_layout_passes=False)

    @functools.partial(
        pl.kernel,
        compiler_params=cp,
        out_type=jax.ShapeDtypeStruct((NC, n, d), jnp.float32),
        mesh=mesh,
        scratch_types=[
            pltpu.VMEM((ep,), jnp.int32),
            pltpu.VMEM((ep,), jnp.int32),
            *[pltpu.VMEM((chunk, d), jnp.float32) for _ in range(nb)],
            *[pltpu.VMEM((chunk,), jnp.float32) for _ in range(nb)],
            pltpu.VMEM_SHARED((n, d), jnp.float32),
            *[pltpu.SemaphoreType.DMA for _ in range(2 + 2 * nb)],
        ],
    )
    def prop(h_hbm, src_hbm, dst_hbm, w_hbm, out_hbm, src_v, dst_v, *rest):
        gbufs = rest[:nb]
        wbufs = rest[nb:2 * nb]
        acc_sh = rest[2 * nb]
        sem_e, sem_z = rest[2 * nb + 1], rest[2 * nb + 2]
        sem_g = rest[2 * nb + 3:2 * nb + 3 + nb]
        sem_s = rest[2 * nb + 3 + nb:]
        cid = lax.axis_index("c")
        sid = lax.axis_index("s")
        wid = sid * NC + cid
        row0 = sid * rpt
        ebase0 = wid * ep

        def g_copy(b, k):
            return pltpu.make_async_copy(
                h_hbm.at[src_v.at[pl.ds(k * chunk, chunk)]], gbufs[b], sem_g[b])

        def w_copy(b, k):
            return pltpu.make_async_copy(
                w_hbm.at[pl.ds(ebase0 + k * chunk, chunk)], wbufs[b], sem_g[b])

        def s_copy(b, k):
            return pltpu.make_async_copy(
                gbufs[b], acc_sh.at[dst_v.at[pl.ds(k * chunk, chunk)]], sem_s[b])

        def g_start(b, k):
            g_copy(b, k).start()
            w_copy(b, k).start()

        def g_wait(b, k):
            g_copy(b, k).wait()
            w_copy(b, k).wait()

        pltpu.async_copy(src_hbm.at[pl.ds(ebase0, ep)], src_v, sem_e)
        pltpu.async_copy(dst_hbm.at[pl.ds(ebase0, ep)], dst_v, sem_e)

        @pl.loop(0, chunk)
        def _(i):
            for j in range(d // L):
                gbufs[0][i, pl.ds(j * L, L)] = jnp.zeros((L,), jnp.float32)

        zn = rpt // chunk
        for t in range(zn):
            pltpu.async_copy(gbufs[0], acc_sh.at[pl.ds(row0 + t * chunk, chunk)],
                             sem_z)
        pltpu.make_async_copy(src_hbm.at[pl.ds(ebase0, ep)], src_v, sem_e).wait()
        pltpu.make_async_copy(dst_hbm.at[pl.ds(ebase0, ep)], dst_v, sem_e).wait()
        for b in range(1, nb):
            g_start(b, b)
        for t in range(zn):
            pltpu.make_async_copy(gbufs[0],
                                  acc_sh.at[pl.ds(row0 + t * chunk, chunk)],
                                  sem_z).wait()
        g_start(0, 0)
        plsc.subcore_barrier()

        @pl.loop(0, ncl, step=nb)
        def _(c0):
            for b in range(nb):
                k = c0 + b
                g_wait(b, k)

                @plsc.parallel_loop(0, chunk, unroll=2)
                def _(i):
                    wrow = plsc.load_gather(
                        wbufs[b], [jnp.full((L,), i, jnp.int32)])
                    for j in range(d // L):
                        sl = pl.ds(j * L, L)
                        gbufs[b][i, sl] = gbufs[b][i, sl] * wrow

                pltpu.async_copy(gbufs[b],
                                 acc_sh.at[dst_v.at[pl.ds(k * chunk, chunk)]],
                                 sem_s[b], add=True)
            for b in range(nb):
                k = c0 + b
                s_copy(b, k).wait()

                @pl.when(c0 + nb < ncl)
                def _():
                    g_start(b, k + nb)

        plsc.subcore_barrier()
        pltpu.sync_copy(acc_sh.at[pl.ds(row0, rpt)],
                        out_hbm.at[cid, pl.ds(row0, rpt)])

    return prop


def _mm1_body(x_ref, w_ref, o_ref):
    o_ref[...] = jnp.dot(x_ref[...], w_ref[...],
                         preferred_element_type=jnp.float32)


def _mid_body(p_ref, b_ref, w_ref, o_ref):
    h = jax.nn.relu(p_ref[0] + p_ref[1] + b_ref[...])
    o_ref[...] = jnp.dot(h, w_ref[...], preferred_element_type=jnp.float32)


def _fin_body(p_ref, b_ref, o_ref):
    c = o_ref.shape[1]
    o_ref[...] = p_ref[0, :, :c] + p_ref[1, :, :c] + b_ref[...]


def kernel(x, edge_index, edge_weight, W1, bias1, W2, bias2):
    n, f_in = x.shape
    hid = W1.shape[1]
    c = W2.shape[1]
    e = edge_weight.shape[0]
    d2 = 128
    np_ = 10240

    chunk, nb = 40, 5
    src, dst = edge_index[0], edge_index[1]
    W2p = jnp.pad(W2, ((0, 0), (0, d2 - c)))
    xp = jnp.pad(x, ((0, np_ - n), (0, 0)))

    bn = 2048
    h1 = pl.pallas_call(
        _mm1_body,
        grid=(np_ // bn,),
        in_specs=[
            pl.BlockSpec((bn, f_in), lambda i: (i, 0)),
            pl.BlockSpec((f_in, hid), lambda i: (0, 0)),
        ],
        out_specs=pl.BlockSpec((bn, hid), lambda i: (i, 0)),
        out_shape=jax.ShapeDtypeStruct((np_, hid), jnp.float32),
    )(xp, W1)

    p1 = _make_propagate(np_, hid, e, chunk, nb)(h1, src, dst, edge_weight)

    h2 = pl.pallas_call(
        _mid_body,
        grid=(np_ // bn,),
        in_specs=[
            pl.BlockSpec((NC, bn, hid), lambda i: (0, i, 0)),
            pl.BlockSpec((hid,), lambda i: (0,)),
            pl.BlockSpec((hid, d2), lambda i: (0, 0)),
        ],
        out_specs=pl.BlockSpec((bn, d2), lambda i: (i, 0)),
        out_shape=jax.ShapeDtypeStruct((np_, d2), jnp.float32),
    )(p1, bias1, W2p)

    p2 = _make_propagate(np_, d2, e, chunk, nb)(h2, src, dst, edge_weight)

    out = pl.pallas_call(
        _fin_body,
        grid=(np_ // bn,),
        in_specs=[
            pl.BlockSpec((NC, bn, d2), lambda i: (0, i, 0)),
            pl.BlockSpec((c,), lambda i: (0,)),
        ],
        out_specs=pl.BlockSpec((bn, c), lambda i: (i, 0)),
        out_shape=jax.ShapeDtypeStruct((np_, c), jnp.float32),
    )(p2, bias2)
    return out[:n]
